# baseline (speedup 1.0000x reference)
import jax
import jax.numpy as jnp
from jax.experimental import pallas as pl
from jax.experimental.pallas import tpu as pltpu

_BATCH = 1024
_N_CONT = 13
_N_CAT = 26
_VOCAB = 1000
_EMB = 50
_TOTAL_IN = _N_CONT + _N_CAT * _VOCAB
_TOTAL_OUT = _N_CONT + _N_CAT * _EMB
_TILE_B = 128
_NCHUNK = 4
_CW = (-(-_TOTAL_IN // 128) * 128) // _NCHUNK
assert _CW % 128 == 0 and _CW * _NCHUNK >= _TOTAL_IN
_WPAD = 1152

_PARTS = []
for _f in range(_N_CAT):
    _s = _N_CONT + _f * _VOCAB
    _e = _s + _VOCAB
    for _k in range(_s // _CW, (_e - 1) // _CW + 1):
        _gs = max(_s, _k * _CW)
        _ge = min(_e, (_k + 1) * _CW)
        _ls = _gs - _k * _CW
        _a = (_ls // 128) * 128
        _off = _ls - _a
        _w = _ge - _gs
        _valid = min(_CW, _TOTAL_IN - _k * _CW)
        _sw = min(-(-(_off + _w) // 128) * 128, _valid - _a)
        _PARTS.append((_f, _k, _a, _sw, _off, _gs - _s, _ge - _s))


def _body(*refs):
    x_refs = refs[:_NCHUNK]
    w_ref = refs[_NCHUNK]
    o_ref = refs[_NCHUNK + 1]
    o_ref[:, :_N_CONT] = x_refs[0][:, :_N_CONT]
    acc = {}
    for p, (f, k, a, sw, _off, _lo, _hi) in enumerate(_PARTS):
        d = jnp.dot(
            x_refs[k][:, a : a + sw],
            w_ref[p, :sw, :],
            preferred_element_type=jnp.float32,
            precision=jax.lax.Precision.DEFAULT,
        )
        acc[f] = d if f not in acc else acc[f] + d
    for f, v in acc.items():
        o_ref[:, _N_CONT + f * _EMB : _N_CONT + (f + 1) * _EMB] = v


def kernel(inputs, embeddings):
    wparts = jnp.stack(
        [
            jnp.pad(
                embeddings[f, lo:hi],
                ((off, _WPAD - off - (hi - lo)), (0, 0)),
            )
            for (f, _k, _a, _sw, off, lo, hi) in _PARTS
        ]
    )

    in_specs = [
        pl.BlockSpec((_TILE_B, _CW), (lambda i, k=k: (i, k)))
        for k in range(_NCHUNK)
    ]
    in_specs.append(
        pl.BlockSpec((len(_PARTS), _WPAD, _EMB), lambda i: (0, 0, 0))
    )

    return pl.pallas_call(
        _body,
        grid=(_BATCH // _TILE_B,),
        in_specs=in_specs,
        out_specs=pl.BlockSpec((_TILE_B, _TOTAL_OUT), lambda i: (i, 0)),
        out_shape=jax.ShapeDtypeStruct((_BATCH, _TOTAL_OUT), jnp.float32),
    )(*([inputs] * _NCHUNK), wparts)


# speedup vs baseline: 1.0813x; 1.0813x over previous
import jax
import jax.numpy as jnp
from jax.experimental import pallas as pl
from jax.experimental.pallas import tpu as pltpu

_BATCH = 1024
_N_CONT = 13
_N_CAT = 26
_VOCAB = 1000
_EMB = 50
_TOTAL_IN = _N_CONT + _N_CAT * _VOCAB
_TOTAL_OUT = _N_CONT + _N_CAT * _EMB
_TILE_B = 64
_WPAD = 1152

_STARTS = [_N_CONT + f * _VOCAB for f in range(_N_CAT)]
_ALIGNED = [(s // 128) * 128 for s in _STARTS]
_OFFS = [s - a for s, a in zip(_STARTS, _ALIGNED)]


def _body(x_ref, w_ref, o_ref):
    o_ref[:, :_N_CONT] = x_ref[:, :_N_CONT]
    for f in range(_N_CAT):
        a = _ALIGNED[f]
        w = min(_WPAD, _TOTAL_IN - a)
        x = x_ref[:, a : a + w]
        o_ref[:, _N_CONT + f * _EMB : _N_CONT + (f + 1) * _EMB] = jnp.dot(
            x,
            w_ref[f, :w, :],
            preferred_element_type=jnp.float32,
            precision=jax.lax.Precision.DEFAULT,
        )


def kernel(inputs, embeddings):
    wshift = jnp.stack(
        [
            jnp.pad(embeddings[f], ((off, _WPAD - _VOCAB - off), (0, 0)))
            for f, off in enumerate(_OFFS)
        ]
    )

    return pl.pallas_call(
        _body,
        grid=(_BATCH // _TILE_B,),
        in_specs=[
            pl.BlockSpec((_TILE_B, _TOTAL_IN), lambda i: (i, 0)),
            pl.BlockSpec((_N_CAT, _WPAD, _EMB), lambda i: (0, 0, 0)),
        ],
        out_specs=pl.BlockSpec((_TILE_B, _TOTAL_OUT), lambda i: (i, 0)),
        out_shape=jax.ShapeDtypeStruct((_BATCH, _TOTAL_OUT), jnp.float32),
    )(inputs, wshift)


# speedup vs baseline: 1.0939x; 1.0116x over previous
import jax
import jax.numpy as jnp
from jax.experimental import pallas as pl
from jax.experimental.pallas import tpu as pltpu

_BATCH = 1024
_N_CONT = 13
_N_CAT = 26
_VOCAB = 1000
_EMB = 50
_TOTAL_IN = _N_CONT + _N_CAT * _VOCAB
_TOTAL_OUT = _N_CONT + _N_CAT * _EMB
_RT = 64
_NSUB = 4
_TILE_B = _RT * _NSUB
_NROW = _BATCH // _TILE_B
_WPAD = 1152

_STARTS = [_N_CONT + f * _VOCAB for f in range(_N_CAT)]
_ALIGNED = [(s // 128) * 128 for s in _STARTS]
_OFFS = [s - a for s, a in zip(_STARTS, _ALIGNED)]


def _body(in_hbm, w_ref, o_ref, slots, sems):
    i = pl.program_id(0)

    def _copy(step, j):
        src = in_hbm.at[pl.ds((step * _NSUB + j) * _RT, _RT), :]
        return pltpu.make_async_copy(src, slots.at[j], sems.at[j])

    @pl.when(i == 0)
    def _():
        for j in range(_NSUB):
            _copy(0, j).start()

    for j in range(_NSUB):
        _copy(i, j).wait()
        x = slots.at[j]
        rows = pl.ds(j * _RT, _RT)
        o_ref[rows, : _N_CONT] = x[:, :_N_CONT]
        for f in range(_N_CAT):
            a = _ALIGNED[f]
            w = min(_WPAD, _TOTAL_IN - a)
            o_ref[rows, _N_CONT + f * _EMB : _N_CONT + (f + 1) * _EMB] = (
                jnp.dot(
                    x[:, a : a + w],
                    w_ref[f, :w, :],
                    preferred_element_type=jnp.float32,
                    precision=jax.lax.Precision.DEFAULT,
                )
            )
        @pl.when(i < _NROW - 1)
        def _():
            _copy(i + 1, j).start()


def kernel(inputs, embeddings):
    wshift = jnp.stack(
        [
            jnp.pad(embeddings[f], ((off, _WPAD - _VOCAB - off), (0, 0)))
            for f, off in enumerate(_OFFS)
        ]
    )

    return pl.pallas_call(
        _body,
        grid=(_NROW,),
        in_specs=[
            pl.BlockSpec(memory_space=pl.ANY),
            pl.BlockSpec((_N_CAT, _WPAD, _EMB), lambda i: (0, 0, 0)),
        ],
        out_specs=pl.BlockSpec((_TILE_B, _TOTAL_OUT), lambda i: (i, 0)),
        out_shape=jax.ShapeDtypeStruct((_BATCH, _TOTAL_OUT), jnp.float32),
        scratch_shapes=[
            pltpu.VMEM((_NSUB, _RT, _TOTAL_IN), jnp.float32),
            pltpu.SemaphoreType.DMA((_NSUB,)),
        ],
        compiler_params=pltpu.CompilerParams(
            dimension_semantics=("arbitrary",),
        ),
    )(inputs, wshift)


# speedup vs baseline: 1.0998x; 1.0053x over previous
import jax
import jax.numpy as jnp
from jax.experimental import pallas as pl
from jax.experimental.pallas import tpu as pltpu

_BATCH = 1024
_N_CONT = 13
_N_CAT = 26
_VOCAB = 1000
_EMB = 50
_TOTAL_IN = _N_CONT + _N_CAT * _VOCAB
_TOTAL_OUT = _N_CONT + _N_CAT * _EMB
_TILE_B = 128
_WPAD = 1152

_STARTS = [_N_CONT + f * _VOCAB for f in range(_N_CAT)]
_ALIGNED = [(s // 128) * 128 for s in _STARTS]
_OFFS = [s - a for s, a in zip(_STARTS, _ALIGNED)]


def _body(x_ref, w_ref, o_ref):
    o_ref[:, :_N_CONT] = x_ref[:, :_N_CONT]
    for f in range(_N_CAT):
        a = _ALIGNED[f]
        w = min(_WPAD, _TOTAL_IN - a)
        x = x_ref[:, a : a + w]
        o_ref[:, _N_CONT + f * _EMB : _N_CONT + (f + 1) * _EMB] = jnp.dot(
            x,
            w_ref[f, :w, :],
            preferred_element_type=jnp.float32,
            precision=jax.lax.Precision.DEFAULT,
        )


def kernel(inputs, embeddings):
    wshift = jnp.stack(
        [
            jnp.pad(embeddings[f], ((off, _WPAD - _VOCAB - off), (0, 0)))
            for f, off in enumerate(_OFFS)
        ]
    )

    return pl.pallas_call(
        _body,
        grid=(_BATCH // _TILE_B,),
        in_specs=[
            pl.BlockSpec((_TILE_B, _TOTAL_IN), lambda i: (i, 0)),
            pl.BlockSpec((_N_CAT, _WPAD, _EMB), lambda i: (0, 0, 0)),
        ],
        out_specs=pl.BlockSpec((_TILE_B, _TOTAL_OUT), lambda i: (i, 0)),
        out_shape=jax.ShapeDtypeStruct((_BATCH, _TOTAL_OUT), jnp.float32),
        compiler_params=pltpu.CompilerParams(
            dimension_semantics=("parallel",),
        ),
    )(inputs, wshift)


# speedup vs baseline: 4.4250x; 4.0236x over previous
import jax
import jax.numpy as jnp
from jax.experimental import pallas as pl
from jax.experimental.pallas import tpu as pltpu

_BATCH = 1024
_N_CONT = 13
_N_CAT = 26
_VOCAB = 1000
_EMB = 50
_TOTAL_IN = _N_CONT + _N_CAT * _VOCAB
_TOTAL_OUT = _N_CONT + _N_CAT * _EMB
_TILE_B = 128
_SHIFT = _N_CONT % 8
_WPAD = _VOCAB + 8

_STARTS = [_N_CONT + f * _VOCAB for f in range(_N_CAT)]
_ALIGNED = [s - _SHIFT for s in _STARTS]


def _body(x_ref, w_ref, o_ref):
    o_ref[:_N_CONT, :] = x_ref[:_N_CONT, :]
    for f in range(_N_CAT):
        a = _ALIGNED[f]
        w = min(_WPAD, _TOTAL_IN - a)
        o_ref[_N_CONT + f * _EMB : _N_CONT + (f + 1) * _EMB, :] = jnp.dot(
            w_ref[f, :, :w],
            x_ref[a : a + w, :],
            preferred_element_type=jnp.float32,
            precision=jax.lax.Precision.DEFAULT,
        )


def kernel(inputs, embeddings):
    xt = inputs.T
    wt = embeddings.transpose(0, 2, 1)
    w2 = jnp.pad(wt, ((0, 0), (0, 0), (_SHIFT, _WPAD - _VOCAB - _SHIFT)))

    out_t = pl.pallas_call(
        _body,
        grid=(_BATCH // _TILE_B,),
        in_specs=[
            pl.BlockSpec((_TOTAL_IN, _TILE_B), lambda i: (0, i)),
            pl.BlockSpec((_N_CAT, _EMB, _WPAD), lambda i: (0, 0, 0)),
        ],
        out_specs=pl.BlockSpec((_TOTAL_OUT, _TILE_B), lambda i: (0, i)),
        out_shape=jax.ShapeDtypeStruct((_TOTAL_OUT, _BATCH), jnp.float32),
    )(xt, w2)
    return out_t.T
